# baseline (speedup 1.0000x reference)
import functools

import jax
import jax.numpy as jnp
from jax import lax
from jax.experimental import pallas as pl
from jax.experimental.pallas import tpu as pltpu
from jax.experimental.pallas import tpu_sc as plsc

_VOCAB = 1_000_000
_BATCH, _SEQ, _D = 4096, 200, 32
_DP = 128
_NC, _NS = 2, 16
_NW = _NC * _NS
_BW = _BATCH // _NW
_SP = 2
_NIT = _SEQ // _SP
_G = _BW // 16
_TOK = _SP * _BW
_SKEW = _DP + 1

_mesh = plsc.VectorSubcoreMesh(core_axis_name="c", subcore_axis_name="s")


@functools.partial(
    pl.kernel,
    out_type=jax.ShapeDtypeStruct((_SEQ, _D, _BATCH), jnp.float32),
    mesh=_mesh,
    scratch_types=[
        pltpu.VMEM((_SP, _BW), jnp.int32),
        pltpu.VMEM((_SP, _BW), jnp.int32),
        pltpu.VMEM((_TOK,), jnp.int32),
        pltpu.VMEM((_TOK,), jnp.int32),
        pltpu.VMEM((_TOK, _DP), jnp.float32),
        pltpu.VMEM((_TOK, _DP), jnp.float32),
        pltpu.VMEM((_TOK * _SKEW,), jnp.float32),
        pltpu.VMEM((_SP, _D, _BW), jnp.float32),
        pltpu.VMEM((_SP, _D, _BW), jnp.float32),
        pltpu.SemaphoreType.DMA,
        pltpu.SemaphoreType.DMA,
        pltpu.SemaphoreType.DMA,
        pltpu.SemaphoreType.DMA,
    ],
    compiler_params=pltpu.CompilerParams(use_tc_tiling_on_sc=True,
                                         needs_layout_passes=False),
)
def _gather_kernel(ids_hbm, table_hbm, out_hbm,
                   ix0, ix1, ig0, ig1, rw0, rw1, skew, ob0, ob1,
                   sg0, sg1, ss0, ss1):
    wid = lax.axis_index("s") * _NC + lax.axis_index("c")
    bo = wid * _BW
    ixs, igs = [ix0, ix1], [ig0, ig1]
    rws, obs = [rw0, rw1], [ob0, ob1]
    sgs, sss = [sg0, sg1], [ss0, ss1]
    iota16 = jax.lax.iota(jnp.int32, 16)
    iota_skew = iota16 * _SKEW

    def fetch(t, p):
        pltpu.sync_copy(ids_hbm.at[pl.ds(t * _SP, _SP), pl.ds(bo, _BW)],
                        ixs[p])
        for a in range(_SP):
            for g in range(_G):
                v = ixs[p][a, pl.ds(g * 16, 16)]
                igs[p][pl.ds(a * _BW + g * 16, 16)] = (
                    lax.shift_right_logical(v, 2))
        pltpu.async_copy(table_hbm.at[igs[p]], rws[p], sgs[p])

    fetch(0, 0)
    fetch(1, 1)

    @pl.loop(0, _NIT, step=2)
    def _(tt):
        for p in range(2):
            t = tt + p
            pltpu.make_async_copy(table_hbm.at[igs[p]],
                                  rws[p], sgs[p]).wait()

            @pl.when(t >= 2)
            def _():
                pltpu.make_async_copy(
                    obs[p], out_hbm.at[pl.ds(0, _SP), :, pl.ds(bo, _BW)],
                    sss[p]).wait()

            @plsc.parallel_loop(0, _TOK, unroll=2)
            def _(tok):
                for h in range(_DP // 16):
                    skew[pl.ds(tok * _SKEW + h * 16, 16)] = (
                        rws[p][tok, pl.ds(h * 16, 16)])

            for a in range(_SP):
                @pl.loop(0, _G)
                def _(g):
                    lane0 = (ixs[p][a, pl.ds(g * 16, 16)] & 3) * 32
                    tvec = (a * _BW + g * 16) * _SKEW + iota_skew + lane0
                    for j in range(_D):
                        vals = plsc.load_gather(skew, [tvec + j])
                        obs[p][a, j, pl.ds(g * 16, 16)] = vals

            @pl.when(t + 2 < _NIT)
            def _():
                fetch(t + 2, p)

            pltpu.async_copy(
                obs[p],
                out_hbm.at[pl.ds(t * _SP, _SP), :, pl.ds(bo, _BW)],
                sss[p])

    for p in range(2):
        pltpu.make_async_copy(
            obs[p], out_hbm.at[pl.ds(0, _SP), :, pl.ds(bo, _BW)],
            sss[p]).wait()


def kernel(input_ids, embedding):
    ids_t = input_ids.T
    table128 = jnp.concatenate(
        [embedding[q::4] for q in range(4)], axis=1)
    out = _gather_kernel(ids_t, table128)
    return out.transpose(2, 0, 1)


# speedup vs baseline: 5.5102x; 5.5102x over previous
import functools

import jax
import jax.numpy as jnp
from jax import lax
from jax.experimental import pallas as pl
from jax.experimental.pallas import tpu as pltpu
from jax.experimental.pallas import tpu_sc as plsc

_VOCAB = 1_000_000
_BATCH, _SEQ, _D = 4096, 200, 32
_DP = 128
_NC, _NS = 2, 16
_NW = _NC * _NS
_BW = _BATCH // _NW
_SP = 2
_NIT = _SEQ // _SP
_G = _BW // 16
_TOK = _SP * _BW
_SKEW = _DP + 1

_mesh = plsc.VectorSubcoreMesh(core_axis_name="c", subcore_axis_name="s")


@functools.partial(
    pl.kernel,
    out_type=jax.ShapeDtypeStruct((_SEQ, _D, _BATCH), jnp.float32),
    mesh=_mesh,
    scratch_types=[
        pltpu.VMEM((_SP, _BW), jnp.int32),
        pltpu.VMEM((_SP, _BW), jnp.int32),
        pltpu.VMEM((_TOK,), jnp.int32),
        pltpu.VMEM((_TOK,), jnp.int32),
        pltpu.VMEM((_TOK, _DP), jnp.float32),
        pltpu.VMEM((_TOK, _DP), jnp.float32),
        pltpu.VMEM((_TOK * _SKEW,), jnp.float32),
        pltpu.VMEM((_SP, _D, _BW), jnp.float32),
        pltpu.VMEM((_SP, _D, _BW), jnp.float32),
        pltpu.SemaphoreType.DMA,
        pltpu.SemaphoreType.DMA,
        pltpu.SemaphoreType.DMA,
        pltpu.SemaphoreType.DMA,
    ],
    compiler_params=pltpu.CompilerParams(use_tc_tiling_on_sc=True,
                                         needs_layout_passes=False),
)
def _gather_kernel(ids_hbm, table_hbm, out_hbm,
                   ix0, ix1, ig0, ig1, rw0, rw1, skew, ob0, ob1,
                   sg0, sg1, ss0, ss1):
    wid = lax.axis_index("s") * _NC + lax.axis_index("c")
    bo = wid * _BW
    ixs, igs = [ix0, ix1], [ig0, ig1]
    rws, obs = [rw0, rw1], [ob0, ob1]
    sgs, sss = [sg0, sg1], [ss0, ss1]
    iota16 = jax.lax.iota(jnp.int32, 16)
    iota_skew = iota16 * _SKEW

    def fetch(t, p):
        pltpu.sync_copy(ids_hbm.at[pl.ds(t * _SP, _SP), pl.ds(bo, _BW)],
                        ixs[p])
        for a in range(_SP):
            for g in range(_G):
                v = ixs[p][a, pl.ds(g * 16, 16)]
                igs[p][pl.ds(a * _BW + g * 16, 16)] = (
                    lax.shift_right_logical(v, 2))
        pltpu.async_copy(table_hbm.at[igs[p]], rws[p], sgs[p])

    fetch(0, 0)
    fetch(1, 1)

    @pl.loop(0, _NIT, step=2)
    def _(tt):
        for p in range(2):
            t = tt + p
            pltpu.make_async_copy(table_hbm.at[igs[p]],
                                  rws[p], sgs[p]).wait()

            @pl.when(t >= 2)
            def _():
                pltpu.make_async_copy(
                    obs[p], out_hbm.at[pl.ds(0, _SP), :, pl.ds(bo, _BW)],
                    sss[p]).wait()

            @plsc.parallel_loop(0, _TOK, unroll=2)
            def _(tok):
                for h in range(_DP // 16):
                    skew[pl.ds(tok * _SKEW + h * 16, 16)] = (
                        rws[p][tok, pl.ds(h * 16, 16)])

            for a in range(_SP):
                @pl.loop(0, _G)
                def _(g):
                    lane0 = (ixs[p][a, pl.ds(g * 16, 16)] & 3) * 32
                    tvec = (a * _BW + g * 16) * _SKEW + iota_skew + lane0
                    for j in range(_D):
                        vals = plsc.load_gather(skew, [tvec + j])
                        obs[p][a, j, pl.ds(g * 16, 16)] = vals

            @pl.when(t + 2 < _NIT)
            def _():
                fetch(t + 2, p)

            pltpu.async_copy(
                obs[p],
                out_hbm.at[pl.ds(t * _SP, _SP), :, pl.ds(bo, _BW)],
                sss[p])

    for p in range(2):
        pltpu.make_async_copy(
            obs[p], out_hbm.at[pl.ds(0, _SP), :, pl.ds(bo, _BW)],
            sss[p]).wait()


def kernel(input_ids, embedding):
    ids_t = input_ids.T
    table128 = embedding.reshape(_VOCAB // 4, _DP)
    out = _gather_kernel(ids_t, table128)
    return out.transpose(2, 0, 1)
